# baseline (speedup 1.0000x reference)
import functools
import math

import jax
import jax.numpy as jnp
from jax.experimental import pallas as pl
from jax.experimental.pallas import tpu as pltpu

_N = 10000
_NP = 10240
_F = 128


def _mm_kernel(a_ref, b_ref, o_ref, acc_ref, *, nk, zero_diag, nt):
    kk = pl.program_id(2)

    @pl.when(kk == 0)
    def _init():
        acc_ref[...] = jnp.zeros_like(acc_ref)

    a = a_ref[...]
    b = b_ref[...]
    if a.dtype != b.dtype:
        a = a.astype(jnp.float32)
        b = b.astype(jnp.float32)
    if nt:
        acc_ref[...] += jax.lax.dot_general(
            a, b, (((1,), (1,)), ((), ())), preferred_element_type=jnp.float32)
    else:
        acc_ref[...] += jnp.dot(a, b, preferred_element_type=jnp.float32)

    @pl.when(kk == nk - 1)
    def _done():
        out = acc_ref[...]
        if zero_diag:
            i = pl.program_id(0)
            j = pl.program_id(1)
            rr = jax.lax.broadcasted_iota(jnp.int32, out.shape, 0)
            cc = jax.lax.broadcasted_iota(jnp.int32, out.shape, 1)
            out = jnp.where(jnp.logical_and(i == j, rr == cc), 0.0, out)
        o_ref[...] = out.astype(o_ref.dtype)


def _mm(a, b, *, nt=False, zero_diag=False, out_dtype=jnp.float32):
    m, k = a.shape
    n = b.shape[0] if nt else b.shape[1]
    bm = 1280 if m % 1280 == 0 else m
    bk = 1280 if k % 1280 == 0 else k
    bn = 1280 if n % 1280 == 0 else n
    nk = k // bk
    grid = (m // bm, n // bn, nk)
    if nt:
        in_specs = [pl.BlockSpec((bm, bk), lambda i, j, q: (i, q)),
                    pl.BlockSpec((bn, bk), lambda i, j, q: (j, q))]
    else:
        in_specs = [pl.BlockSpec((bm, bk), lambda i, j, q: (i, q)),
                    pl.BlockSpec((bk, bn), lambda i, j, q: (q, j))]
    return pl.pallas_call(
        functools.partial(_mm_kernel, nk=nk, zero_diag=zero_diag, nt=nt),
        grid=grid,
        in_specs=in_specs,
        out_specs=pl.BlockSpec((bm, bn), lambda i, j, q: (i, j)),
        out_shape=jax.ShapeDtypeStruct((m, n), out_dtype),
        scratch_shapes=[pltpu.VMEM((bm, bn), jnp.float32)],
        compiler_params=pltpu.CompilerParams(
            dimension_semantics=("parallel", "parallel", "arbitrary")),
    )(a, b)


def _dinv(deg):
    return jnp.where(deg > 0.0, 1.0 / jnp.sqrt(jnp.maximum(deg, 1e-12)), 0.0)


def _gcn_pooled(h, A, W, b):
    deg = jnp.sum(A, axis=1, dtype=jnp.float32) + 2.0
    di = _dinv(deg)
    u = di[:, None] * _mm(h, W)
    Av = _mm(A, u)
    return di[:, None] * (Av + 2.0 * u) + b


def _pool_scores(h, pw, n_real):
    s = jnp.tanh(jnp.dot(h, pw) / jnp.linalg.norm(pw))
    return jnp.where(jnp.arange(h.shape[0]) < n_real, s, -2.0)


def _pad_gather(h, perm, vals, kpad):
    k = perm.shape[0]
    idxp = jnp.zeros((kpad,), jnp.int32).at[:k].set(perm)
    valsp = jnp.zeros((kpad,), jnp.float32).at[:k].set(vals)
    return h[idxp] * valsp[:, None]


def _sel_rows(A, At, perm, kpad, dtype):
    k = perm.shape[0]
    idxp = jnp.zeros((kpad,), jnp.int32).at[:k].set(perm)
    valid = jnp.arange(kpad) < k
    r = jnp.arange(k)
    G = jnp.where(valid[:, None], A[idxp, :], 0).astype(dtype)
    G = G.at[r, perm].set(1.0)
    H = jnp.where(valid[:, None], At[idxp, :], 0).astype(dtype)
    H = H.at[r, perm].set(1.0)
    return G, H


def kernel(x, edge_index, down_W0, down_b0, down_W1, down_b1, down_W2,
           down_b2, down_W3, down_b3, up_W0, up_b0, up_W1, up_b1, up_W2,
           up_b2, pool_w0, pool_w1, pool_w2):
    src = edge_index[0]
    dst = edge_index[1]
    ones_e = jnp.ones(src.shape, jnp.bfloat16)

    A0 = jnp.zeros((_NP, _NP), jnp.bfloat16).at[dst, src].add(ones_e)
    deg_in = jnp.zeros((_NP,), jnp.float32).at[dst].add(1.0)
    c0 = jnp.zeros((_NP,), jnp.float32).at[dst].add(
        (src == dst).astype(jnp.float32))
    s0 = jnp.where(c0 == 0.0, 2.0, 0.0)
    di0 = _dinv(deg_in + s0)

    xp = jnp.zeros((_NP, _F), jnp.float32).at[:_N].set(x)

    u = di0[:, None] * _mm(xp, down_W0)
    h0 = di0[:, None] * (_mm(A0, u) + s0[:, None] * u) + down_b0
    h0 = jax.nn.relu(h0)

    k1, k1p = 5000, 5120
    sc1 = _pool_scores(h0, pool_w0, _N)
    vals1, perm1 = jax.lax.top_k(sc1, k1)
    hp1 = _pad_gather(h0, perm1, vals1, k1p)
    inv1 = jnp.full((_NP,), k1p, jnp.int32).at[perm1].set(
        jnp.arange(k1, dtype=jnp.int32))
    r1 = jnp.arange(k1)
    G1 = jnp.zeros((k1p, _NP), jnp.bfloat16).at[inv1[dst], src].add(ones_e)
    G1 = G1.at[r1, perm1].set(1.0)
    H1 = jnp.zeros((k1p, _NP), jnp.bfloat16).at[inv1[src], dst].add(ones_e)
    H1 = H1.at[r1, perm1].set(1.0)
    A1 = _mm(G1, H1, nt=True, zero_diag=True, out_dtype=jnp.bfloat16)
    h1 = jax.nn.relu(_gcn_pooled(hp1, A1, down_W1, down_b1))

    k2, k2p = 2500, 2560
    A1t = A1.T
    sc2 = _pool_scores(h1, pool_w1, k1)
    vals2, perm2 = jax.lax.top_k(sc2, k2)
    hp2 = _pad_gather(h1, perm2, vals2, k2p)
    G2, H2 = _sel_rows(A1, A1t, perm2, k2p, jnp.bfloat16)
    A2 = _mm(G2, H2, nt=True, zero_diag=True)
    h2 = jax.nn.relu(_gcn_pooled(hp2, A2, down_W2, down_b2))

    k3, k3p = 1250, 1280
    A2t = A2.T
    sc3 = _pool_scores(h2, pool_w2, k2)
    vals3, perm3 = jax.lax.top_k(sc3, k3)
    hp3 = _pad_gather(h2, perm3, vals3, k3p)
    G3, H3 = _sel_rows(A2, A2t, perm3, k3p, jnp.float32)
    A3 = _mm(G3, H3, nt=True, zero_diag=True)
    h3 = jax.nn.relu(_gcn_pooled(hp3, A3, down_W3, down_b3))

    u0 = h2 + jnp.zeros_like(h2).at[perm3].set(h3[:k3])
    g = jax.nn.relu(_gcn_pooled(u0, A2, up_W0, up_b0))

    u1 = h1 + jnp.zeros_like(h1).at[perm2].set(g[:k2])
    g = jax.nn.relu(_gcn_pooled(u1, A1, up_W1, up_b1))

    u2 = h0 + jnp.zeros_like(h0).at[perm1].set(g[:k1])
    v = di0[:, None] * _mm(u2, up_W2)
    out = di0[:, None] * (_mm(A0, v) + s0[:, None] * v) + up_b2

    return out[:_N]


# speedup vs baseline: 3.0161x; 3.0161x over previous
import functools
import math

import jax
import jax.numpy as jnp
from jax.experimental import pallas as pl
from jax.experimental.pallas import tpu as pltpu

_N = 10000
_NP = 10240
_F = 128


def _mm_kernel(a_ref, b_ref, o_ref, acc_ref, *, nk, zero_diag, nt):
    kk = pl.program_id(2)

    @pl.when(kk == 0)
    def _init():
        acc_ref[...] = jnp.zeros_like(acc_ref)

    a = a_ref[...]
    b = b_ref[...]
    if a.dtype != b.dtype:
        a = a.astype(jnp.float32)
        b = b.astype(jnp.float32)
    if nt:
        acc_ref[...] += jax.lax.dot_general(
            a, b, (((1,), (1,)), ((), ())), preferred_element_type=jnp.float32)
    else:
        acc_ref[...] += jnp.dot(a, b, preferred_element_type=jnp.float32)

    @pl.when(kk == nk - 1)
    def _done():
        out = acc_ref[...]
        if zero_diag:
            i = pl.program_id(0)
            j = pl.program_id(1)
            rr = jax.lax.broadcasted_iota(jnp.int32, out.shape, 0)
            cc = jax.lax.broadcasted_iota(jnp.int32, out.shape, 1)
            out = jnp.where(jnp.logical_and(i == j, rr == cc), 0.0, out)
        o_ref[...] = out.astype(o_ref.dtype)


def _mm(a, b, *, nt=False, zero_diag=False, out_dtype=jnp.float32):
    m, k = a.shape
    n = b.shape[0] if nt else b.shape[1]
    bm = 1280 if m % 1280 == 0 else m
    bk = 1280 if k % 1280 == 0 else k
    bn = 1280 if n % 1280 == 0 else n
    nk = k // bk
    grid = (m // bm, n // bn, nk)
    if nt:
        in_specs = [pl.BlockSpec((bm, bk), lambda i, j, q: (i, q)),
                    pl.BlockSpec((bn, bk), lambda i, j, q: (j, q))]
    else:
        in_specs = [pl.BlockSpec((bm, bk), lambda i, j, q: (i, q)),
                    pl.BlockSpec((bk, bn), lambda i, j, q: (q, j))]
    return pl.pallas_call(
        functools.partial(_mm_kernel, nk=nk, zero_diag=zero_diag, nt=nt),
        grid=grid,
        in_specs=in_specs,
        out_specs=pl.BlockSpec((bm, bn), lambda i, j, q: (i, j)),
        out_shape=jax.ShapeDtypeStruct((m, n), out_dtype),
        scratch_shapes=[pltpu.VMEM((bm, bn), jnp.float32)],
        compiler_params=pltpu.CompilerParams(
            dimension_semantics=("parallel", "parallel", "arbitrary")),
    )(a, b)


def _fix_kernel(g_ref, p_ref, o_ref, *, k):
    i = pl.program_id(0)
    bm, n = o_ref.shape
    g = g_ref[...]
    rows = i * bm + jax.lax.broadcasted_iota(jnp.int32, (bm, n), 0)
    cols = jax.lax.broadcasted_iota(jnp.int32, (bm, n), 1)
    p = p_ref[...]
    out = jnp.where(cols == p, 1.0, g)
    out = jnp.where(rows < k, out, 0.0)
    o_ref[...] = out.astype(o_ref.dtype)


def _fix(g, idxp, k, out_dtype):
    m, n = g.shape
    bm = 256
    return pl.pallas_call(
        functools.partial(_fix_kernel, k=k),
        grid=(m // bm,),
        in_specs=[pl.BlockSpec((bm, n), lambda i: (i, 0)),
                  pl.BlockSpec((bm, 1), lambda i: (i, 0))],
        out_specs=pl.BlockSpec((bm, n), lambda i: (i, 0)),
        out_shape=jax.ShapeDtypeStruct((m, n), out_dtype),
    )(g, idxp.reshape(m, 1))


def _dinv(deg):
    return jnp.where(deg > 0.0, 1.0 / jnp.sqrt(jnp.maximum(deg, 1e-12)), 0.0)


def _gcn_pooled(h, A, W, b):
    deg = jnp.sum(A, axis=1, dtype=jnp.float32) + 2.0
    di = _dinv(deg)
    u = di[:, None] * _mm(h, W)
    Av = _mm(A, u)
    return di[:, None] * (Av + 2.0 * u) + b


def _pool_scores(h, pw, n_real):
    s = jnp.tanh(jnp.dot(h, pw) / jnp.linalg.norm(pw))
    return jnp.where(jnp.arange(h.shape[0]) < n_real, s, -2.0)


def _pad_gather(h, perm, vals, kpad):
    k = perm.shape[0]
    idxp = jnp.zeros((kpad,), jnp.int32).at[:k].set(perm)
    valsp = jnp.zeros((kpad,), jnp.float32).at[:k].set(vals)
    return h[idxp] * valsp[:, None]


def _sel_rows(A, At, perm, kpad, dtype):
    k = perm.shape[0]
    idxp = jnp.zeros((kpad,), jnp.int32).at[:k].set(perm)
    G = _fix(A[idxp, :], idxp, k, dtype)
    H = _fix(At[idxp, :], idxp, k, dtype)
    return G, H


def kernel(x, edge_index, down_W0, down_b0, down_W1, down_b1, down_W2,
           down_b2, down_W3, down_b3, up_W0, up_b0, up_W1, up_b1, up_W2,
           up_b2, pool_w0, pool_w1, pool_w2):
    src = edge_index[0]
    dst = edge_index[1]

    A0 = jnp.zeros((_NP, _NP), jnp.float32).at[dst, src].add(1.0)
    A0t = jnp.zeros((_NP, _NP), jnp.float32).at[src, dst].add(1.0)
    deg_in = jnp.zeros((_NP,), jnp.float32).at[dst].add(1.0)
    c0 = jnp.zeros((_NP,), jnp.float32).at[dst].add(
        (src == dst).astype(jnp.float32))
    s0 = jnp.where(c0 == 0.0, 2.0, 0.0)
    di0 = _dinv(deg_in + s0)

    xp = jnp.zeros((_NP, _F), jnp.float32).at[:_N].set(x)

    u = di0[:, None] * _mm(xp, down_W0)
    h0 = di0[:, None] * (_mm(A0, u) + s0[:, None] * u) + down_b0
    h0 = jax.nn.relu(h0)

    k1, k1p = 5000, 5120
    sc1 = _pool_scores(h0, pool_w0, _N)
    vals1, perm1 = jax.lax.top_k(sc1, k1)
    hp1 = _pad_gather(h0, perm1, vals1, k1p)
    G1, H1 = _sel_rows(A0, A0t, perm1, k1p, jnp.bfloat16)
    A1 = _mm(G1, H1, nt=True, zero_diag=True)
    h1 = jax.nn.relu(_gcn_pooled(hp1, A1, down_W1, down_b1))

    k2, k2p = 2500, 2560
    A1t = A1.T
    sc2 = _pool_scores(h1, pool_w1, k1)
    vals2, perm2 = jax.lax.top_k(sc2, k2)
    hp2 = _pad_gather(h1, perm2, vals2, k2p)
    G2, H2 = _sel_rows(A1, A1t, perm2, k2p, jnp.bfloat16)
    A2 = _mm(G2, H2, nt=True, zero_diag=True)
    h2 = jax.nn.relu(_gcn_pooled(hp2, A2, down_W2, down_b2))

    k3, k3p = 1250, 1280
    A2t = A2.T
    sc3 = _pool_scores(h2, pool_w2, k2)
    vals3, perm3 = jax.lax.top_k(sc3, k3)
    hp3 = _pad_gather(h2, perm3, vals3, k3p)
    G3, H3 = _sel_rows(A2, A2t, perm3, k3p, jnp.float32)
    A3 = _mm(G3, H3, nt=True, zero_diag=True)
    h3 = jax.nn.relu(_gcn_pooled(hp3, A3, down_W3, down_b3))

    u0 = h2 + jnp.zeros_like(h2).at[perm3].set(h3[:k3])
    g = jax.nn.relu(_gcn_pooled(u0, A2, up_W0, up_b0))

    u1 = h1 + jnp.zeros_like(h1).at[perm2].set(g[:k2])
    g = jax.nn.relu(_gcn_pooled(u1, A1, up_W1, up_b1))

    u2 = h0 + jnp.zeros_like(h0).at[perm1].set(g[:k1])
    v = di0[:, None] * _mm(u2, up_W2)
    out = di0[:, None] * (_mm(A0, v) + s0[:, None] * v) + up_b2

    return out[:_N]
